# Initial kernel scaffold; baseline (speedup 1.0000x reference)
import functools

import jax
import jax.numpy as jnp
from jax import lax
from jax.experimental import pallas as pl
from jax.experimental.pallas import tpu as pltpu
from jax.experimental.pallas import tpu_sc as plsc

NUM_CORES = 2
NUM_SUBCORES = 16
NUM_WORKERS = NUM_CORES * NUM_SUBCORES
CHUNK = 128


def _make_emb(total, n_chunks, per_w, D):
    mesh = plsc.VectorSubcoreMesh(core_axis_name="c", subcore_axis_name="s")

    @functools.partial(
        pl.kernel,
        mesh=mesh,
        out_type=jax.ShapeDtypeStruct((total, D), jnp.float32),
        scratch_types=[
            pltpu.VMEM((n_chunks, CHUNK), jnp.int32),
            pltpu.VMEM((CHUNK, D), jnp.float32),
            pltpu.SemaphoreType.DMA,
        ],
    )
    def emb(table_hbm, idx_hbm, out_hbm, idx_v, rows_v, gsem):
        wid = lax.axis_index("s") * NUM_CORES + lax.axis_index("c")
        base = wid * per_w
        pltpu.sync_copy(idx_hbm.at[wid], idx_v)

        def body(j, carry):
            pltpu.async_copy(table_hbm.at[idx_v.at[j]], rows_v, gsem).wait()
            pltpu.sync_copy(rows_v, out_hbm.at[pl.ds(base + j * CHUNK, CHUNK)])
            return carry

        lax.fori_loop(0, n_chunks, body, 0)

    return emb


def kernel(x, weight):
    B, S = x.shape
    V, D = weight.shape
    total = B * S
    per_w = total // NUM_WORKERS
    n_chunks = per_w // CHUNK
    idx = x.reshape(NUM_WORKERS, n_chunks, CHUNK).astype(jnp.int32)
    out = _make_emb(total, n_chunks, per_w, D)(weight, idx)
    return out.reshape(B, S, D)


# speedup vs baseline: 1.4364x; 1.4364x over previous
import functools

import jax
import jax.numpy as jnp
from jax import lax
from jax.experimental import pallas as pl
from jax.experimental.pallas import tpu as pltpu
from jax.experimental.pallas import tpu_sc as plsc

NUM_CORES = 2
NUM_SUBCORES = 16
NUM_WORKERS = NUM_CORES * NUM_SUBCORES
CHUNK = 128


def _make_emb(total, n_chunks, per_w, D):
    mesh = plsc.VectorSubcoreMesh(core_axis_name="c", subcore_axis_name="s")

    @functools.partial(
        pl.kernel,
        mesh=mesh,
        compiler_params=pltpu.CompilerParams(use_tc_tiling_on_sc=False),
        out_type=jax.ShapeDtypeStruct((total, D), jnp.float32),
        scratch_types=[
            pltpu.VMEM((n_chunks, CHUNK), jnp.int32),
            pltpu.VMEM((CHUNK, D), jnp.float32),
            pltpu.SemaphoreType.DMA,
        ],
    )
    def emb(table_hbm, idx_hbm, out_hbm, idx_v, rows_v, gsem):
        wid = lax.axis_index("s") * NUM_CORES + lax.axis_index("c")
        base = wid * per_w
        pltpu.sync_copy(idx_hbm.at[wid], idx_v)

        def body(j, carry):
            pltpu.async_copy(table_hbm.at[idx_v.at[j]], rows_v, gsem).wait()
            pltpu.sync_copy(rows_v, out_hbm.at[pl.ds(base + j * CHUNK, CHUNK)])
            return carry

        lax.fori_loop(0, n_chunks, body, 0)

    return emb


def kernel(x, weight):
    B, S = x.shape
    V, D = weight.shape
    total = B * S
    per_w = total // NUM_WORKERS
    n_chunks = per_w // CHUNK
    idx = x.reshape(NUM_WORKERS, n_chunks, CHUNK).astype(jnp.int32)
    out = _make_emb(total, n_chunks, per_w, D)(weight, idx)
    return out.reshape(B, S, D)


# speedup vs baseline: 1.5775x; 1.0983x over previous
import functools

import jax
import jax.numpy as jnp
from jax import lax
from jax.experimental import pallas as pl
from jax.experimental.pallas import tpu as pltpu
from jax.experimental.pallas import tpu_sc as plsc

NUM_CORES = 2
NUM_SUBCORES = 16
NUM_WORKERS = NUM_CORES * NUM_SUBCORES
CHUNK = 128
NBUF = 8
KST = 2


def _make_emb(total, n_chunks, per_w, D):
    mesh = plsc.VectorSubcoreMesh(core_axis_name="c", subcore_axis_name="s")

    @functools.partial(
        pl.kernel,
        mesh=mesh,
        compiler_params=pltpu.CompilerParams(use_tc_tiling_on_sc=False),
        out_type=jax.ShapeDtypeStruct((total, D), jnp.float32),
        scratch_types=[
            pltpu.VMEM((n_chunks, CHUNK), jnp.int32),
            pltpu.VMEM((NBUF, CHUNK, D), jnp.float32),
            pltpu.SemaphoreType.DMA,
            pltpu.SemaphoreType.DMA,
        ],
    )
    def emb(table_hbm, idx_hbm, out_hbm, idx_v, rows_v, gsem, ssem):
        wid = lax.axis_index("s") * NUM_CORES + lax.axis_index("c")
        base = wid * per_w

        pltpu.sync_copy(idx_hbm.at[wid], idx_v)

        def gather(j):
            pltpu.async_copy(
                table_hbm.at[idx_v.at[j]], rows_v.at[j % NBUF], gsem
            )

        def wait_gather(j):
            pltpu.make_async_copy(
                table_hbm.at[idx_v.at[j]], rows_v.at[j % NBUF], gsem
            ).wait()

        def store(j):
            pltpu.async_copy(
                rows_v.at[j % NBUF], out_hbm.at[pl.ds(base + j * CHUNK, CHUNK)], ssem
            )

        def wait_store(j):
            pltpu.make_async_copy(
                rows_v.at[j % NBUF], out_hbm.at[pl.ds(base + j * CHUNK, CHUNK)], ssem
            ).wait()

        for g in range(NBUF - KST):
            gather(g)
        for j in range(KST):
            wait_gather(j)
            store(j)
            gather(j + NBUF - KST)

        def body(j, carry):
            wait_gather(j)
            store(j)
            wait_store(j - KST)
            gather(j + NBUF - KST)
            return carry

        lax.fori_loop(KST, n_chunks - (NBUF - KST), body, 0)

        for j in range(n_chunks - (NBUF - KST), n_chunks):
            wait_gather(j)
            store(j)
            wait_store(j - KST)
        for j in range(n_chunks - KST, n_chunks):
            wait_store(j)

    return emb


def kernel(x, weight):
    B, S = x.shape
    V, D = weight.shape
    total = B * S
    per_w = total // NUM_WORKERS
    n_chunks = per_w // CHUNK
    idx = x.reshape(NUM_WORKERS, n_chunks, CHUNK).astype(jnp.int32)
    out = _make_emb(total, n_chunks, per_w, D)(weight, idx)
    return out.reshape(B, S, D)


# speedup vs baseline: 2.0013x; 1.2686x over previous
import functools

import jax
import jax.numpy as jnp
from jax import lax
from jax.experimental import pallas as pl
from jax.experimental.pallas import tpu as pltpu
from jax.experimental.pallas import tpu_sc as plsc

NUM_CORES = 2
NUM_SUBCORES = 16
NUM_WORKERS = NUM_CORES * NUM_SUBCORES
CHUNK = 128
GDEPTH = 6


def _make_emb(total, n_chunks, per_w, D, BBLK):
    mesh = plsc.VectorSubcoreMesh(core_axis_name="c", subcore_axis_name="s")
    S = 26
    CB, CI = D // 8, 8

    @functools.partial(
        pl.kernel,
        mesh=mesh,
        compiler_params=pltpu.CompilerParams(
            use_tc_tiling_on_sc=False, needs_layout_passes=False
        ),
        out_type=jax.ShapeDtypeStruct((S, CB, BBLK, CI, CHUNK), jnp.float32),
        scratch_types=[
            pltpu.VMEM((n_chunks, CHUNK), jnp.int32),
            pltpu.VMEM((GDEPTH, CHUNK, D), jnp.float32),
            pltpu.VMEM((2, CB, CI, CHUNK + 1), jnp.float32),
            pltpu.SemaphoreType.DMA,
            pltpu.SemaphoreType.DMA,
        ],
    )
    def emb(table_hbm, idx_hbm, out_hbm, idx_v, rows_v, tile_v, gsem, ssem):
        wid = lax.axis_index("s") * NUM_CORES + lax.axis_index("c")
        base = wid * n_chunks

        pltpu.sync_copy(idx_hbm.at[wid], idx_v)

        lanes = lax.iota(jnp.int32, 16)
        cb_vecs = [(lanes + p * 16) // 8 for p in range(2)]
        ci_vecs = [(lanes + p * 16) % 8 for p in range(2)]

        def gather(j):
            pltpu.async_copy(table_hbm.at[idx_v.at[j]], rows_v.at[j % GDEPTH], gsem)

        def wait_gather(j):
            pltpu.make_async_copy(
                table_hbm.at[idx_v.at[j]], rows_v.at[j % GDEPTH], gsem
            ).wait()

        def transpose(j):
            rslot, tslot = j % GDEPTH, j % 2

            def tbody(bi0, carry):
                for dbi in range(8):
                    bi = bi0 + dbi
                    bi_vec = jnp.full((16,), 0, jnp.int32) + bi
                    for p in range(2):
                        val = rows_v[rslot, bi, pl.ds(16 * p, 16)]
                        plsc.store_scatter(
                            tile_v.at[tslot], [cb_vecs[p], ci_vecs[p], bi_vec], val
                        )
                return carry

            lax.fori_loop(0, CHUNK // 8, lambda i, c: tbody(i * 8, c), 0)

        def store(j):
            t = base + j
            s, bb = t // BBLK, t % BBLK
            pltpu.async_copy(
                tile_v.at[j % 2, :, :, pl.ds(0, CHUNK)], out_hbm.at[s, :, bb], ssem
            )

        def wait_store(j):
            t = base + j
            s, bb = t // BBLK, t % BBLK
            pltpu.make_async_copy(
                tile_v.at[j % 2, :, :, pl.ds(0, CHUNK)], out_hbm.at[s, :, bb], ssem
            ).wait()

        for g in range(GDEPTH):
            gather(g)
        wait_gather(0)
        transpose(0)
        store(0)
        gather(GDEPTH)

        def body(j, carry):
            wait_gather(j)
            transpose(j)
            store(j)
            wait_store(j - 1)
            gather(j + GDEPTH)
            return carry

        lax.fori_loop(1, n_chunks - GDEPTH, body, 0)

        for j in range(n_chunks - GDEPTH, n_chunks):
            wait_gather(j)
            transpose(j)
            store(j)
            wait_store(j - 1)
        wait_store(n_chunks - 1)

    return emb


def kernel(x, weight):
    B, S = x.shape
    V, D = weight.shape
    total = B * S
    per_w = total // NUM_WORKERS
    n_chunks = per_w // CHUNK
    BBLK = B // CHUNK
    idx = jnp.swapaxes(x, 0, 1).reshape(NUM_WORKERS, n_chunks, CHUNK).astype(jnp.int32)
    out5 = _make_emb(total, n_chunks, per_w, D, BBLK)(weight, idx)
    return out5.transpose(2, 4, 0, 1, 3).reshape(B, S, D)
